# baseline (speedup 1.0000x reference)
import functools

import jax
import jax.numpy as jnp
from jax import lax
from jax.experimental import pallas as pl
from jax.experimental.pallas import tpu as pltpu
from jax.experimental.pallas import tpu_sc as plsc

NC = 2
NS = 16
NW = NC * NS
LANES = 16
BLK = 128


def _make_sc_kernel(batch, seq_len, dim):
    assert batch == NW * BLK and dim % LANES == 0
    n_vregs = dim // LANES

    mesh = plsc.VectorSubcoreMesh(core_axis_name="c", subcore_axis_name="s")

    @functools.partial(
        pl.kernel,
        out_type=jax.ShapeDtypeStruct((seq_len * dim, batch), jnp.float32),
        mesh=mesh,
        scratch_types=[
            [pltpu.VMEM((1, BLK), jnp.int32) for _ in range(4)],
            [pltpu.VMEM((BLK, 2 * dim), jnp.float32) for _ in range(4)],
            [pltpu.VMEM((dim, BLK), jnp.float32) for _ in range(2)],
            pltpu.VMEM((seq_len, dim), jnp.float32),
            [pltpu.SemaphoreType.DMA for _ in range(4)],
            [pltpu.SemaphoreType.DMA for _ in range(4)],
            [pltpu.SemaphoreType.DMA for _ in range(2)],
        ],
        compiler_params=pltpu.CompilerParams(
            use_tc_tiling_on_sc=True, needs_layout_passes=False),
    )
    def sc_kernel(idx_hbm, tok_hbm, pos_hbm, out_hbm,
                  idx, rows, out_t, pos_v, isem, gsem, ssem):
        wid = lax.axis_index("s") * NC + lax.axis_index("c")
        b0 = wid * BLK

        pltpu.sync_copy(pos_hbm, pos_v)

        def idx_start(s, p):
            pltpu.async_copy(idx_hbm.at[pl.ds(s, 1), pl.ds(b0, BLK)],
                             idx[p], isem[p])

        def idx_wait(s, p):
            pltpu.make_async_copy(idx_hbm.at[pl.ds(s, 1), pl.ds(b0, BLK)],
                                  idx[p], isem[p]).wait()

        def fetch_start(s, p):
            idx_wait(s, p)
            pltpu.async_copy(tok_hbm.at[idx[p].at[0]], rows[p], gsem[p])

        def fetch_wait(p):
            pltpu.make_async_copy(tok_hbm.at[idx[p].at[0]], rows[p],
                                  gsem[p]).wait()

        def store_start(s, p):
            pltpu.async_copy(
                out_t[p], out_hbm.at[pl.ds(s * dim, dim), pl.ds(b0, BLK)], ssem[p])

        def store_wait(s, p):
            pltpu.make_async_copy(
                out_t[p], out_hbm.at[pl.ds(s * dim, dim), pl.ds(b0, BLK)],
                ssem[p]).wait()

        def process(s, q):
            p, po = q, q % 2
            fetch_wait(p)
            s_splat = jnp.full((LANES,), s, jnp.int32)
            iota = jnp.arange(LANES, dtype=jnp.int32)
            jvecs = [iota + g * LANES for g in range(BLK // LANES)]

            def k_body(k, carry):
                rot = (iota + k) & (LANES - 1)
                for c in range(n_vregs):
                    dvec = rot + c * LANES
                    pd = plsc.load_gather(pos_v, [s_splat, dvec])
                    for g in range(BLK // LANES):
                        val = plsc.load_gather(rows[p], [jvecs[g], dvec]) + pd
                        plsc.store_scatter(out_t[po], [dvec, jvecs[g]], val)
                return carry

            lax.fori_loop(0, LANES, k_body, 0)
            store_start(s, po)

        idx_start(0, 0)
        idx_start(1, 1)
        idx_start(2, 2)
        fetch_start(0, 0)
        fetch_start(1, 1)

        def loop_body(t, carry):
            for q in range(4):
                s = 4 * t + q
                pl.when(s + 3 < seq_len)(
                    functools.partial(idx_start, s + 3, (q + 3) % 4))
                pl.when(s + 2 < seq_len)(
                    functools.partial(fetch_start, s + 2, (q + 2) % 4))
                pl.when(s >= 2)(functools.partial(store_wait, s - 2, q % 2))
                process(s, q)
            return carry

        lax.fori_loop(0, seq_len // 4, loop_body, 0)

        store_wait(seq_len - 2, 0)
        store_wait(seq_len - 1, 1)

    return sc_kernel


def kernel(inputs, token_table, position_table):
    batch, seq_len = inputs.shape
    vocab, dim = token_table.shape
    idx_t = inputs.T
    tok_pad = jnp.pad(token_table, ((0, 0), (0, dim)))
    sc = _make_sc_kernel(batch, seq_len, dim)
    out = sc(idx_t, tok_pad, position_table)
    return out.reshape(seq_len, dim, batch).transpose(2, 0, 1)
